# baseline (speedup 1.0000x reference)
import functools

import jax
import jax.numpy as jnp
from jax import lax
from jax.experimental import pallas as pl
from jax.experimental.pallas import tpu as pltpu
from jax.experimental.pallas import tpu_sc as plsc

N_E = 8192
HALF = N_E // 2
E_DIM = 32
BETA = 0.25
N_TOK = 16384
SPLIT = 1
NTS = N_TOK // SPLIT
T = 1024
NB = NTS // T
NW = 32
CH = NTS // NW
CCH = 8
NCH = CH // CCH
NCH2 = NCH // 2


def _vq_body(zbf_ref, emb_ref, zsum_ref, esum_ref, lc_ref,
             loss_ref, idx_ref, perp_ref, cnt_ref, *, last):
    pid = pl.program_id(0)

    zbf = zbf_ref[...]
    emb2 = emb_ref[...]

    mm2 = jax.lax.dot_general(
        zbf, emb2, (((1,), (1,)), ((), ())),
        preferred_element_type=jnp.float32)

    zs = zsum_ref[0, 0, :][:, None]
    es = esum_ref[...]
    d = (zs + es) - mm2

    col = jax.lax.broadcasted_iota(jnp.int32, (T, HALF), 1)
    d0 = d[:, :HALF]
    d1 = d[:, HALF:]
    m0 = jnp.min(d0, axis=1, keepdims=True)
    m1 = jnp.min(d1, axis=1, keepdims=True)
    m0b = m0.astype(jnp.bfloat16).astype(jnp.float32)
    take1 = m1 < m0b
    dsel = jnp.where(take1, d1, d0)
    msel = jnp.where(take1, m1, m0)
    isel = jnp.min(jnp.where(dsel == msel, col, 2 ** 30), axis=1)
    idx = isel + jnp.where(take1[:, 0], HALF, 0)
    idx_ref[0, 0, :] = idx
    dpick = msel[:, 0]

    hi = idx >> 6
    lo = idx & 63
    ohh = (jax.lax.broadcasted_iota(jnp.int32, (T, 128), 1)
           == hi[:, None]).astype(jnp.float32)
    ohl = (jax.lax.broadcasted_iota(jnp.int32, (T, 64), 1)
           == lo[:, None]).astype(jnp.float32)
    cnt_blk = jax.lax.dot_general(
        ohh, ohl, (((0,), (0,)), ((), ())),
        preferred_element_type=jnp.float32)

    @pl.when(pid == 0)
    def _init():
        loss_ref[...] = lc_ref[0:1, 0:1]
        cnt_ref[...] = lc_ref[:, 1:65]

    loss_ref[...] += jnp.sum(dpick)[None, None]
    cnt_ref[...] += cnt_blk

    if last:
        @pl.when(pid == NB - 1)
        def _fin():
            loss_ref[...] = loss_ref[...] * (BETA / (N_TOK * E_DIM))
            e_mean = cnt_ref[...] * (1.0 / N_TOK)
            ent = jnp.sum(e_mean * jnp.log(e_mean + 1e-10))
            perp_ref[...] = jnp.exp(-ent)[None, None]


def _tc_stage(zbf, emb_w, zsum, esum, lc, *, last):
    return pl.pallas_call(
        functools.partial(_vq_body, last=last),
        grid=(NB,),
        in_specs=[
            pl.BlockSpec((T, E_DIM), lambda i: (i, 0)),
            pl.BlockSpec((N_E, E_DIM), lambda i: (0, 0)),
            pl.BlockSpec((1, 1, T), lambda i: (i, 0, 0)),
            pl.BlockSpec((1, N_E), lambda i: (0, 0)),
            pl.BlockSpec((128, 65), lambda i: (0, 0)),
        ],
        out_specs=[
            pl.BlockSpec((1, 1), lambda i: (0, 0)),
            pl.BlockSpec((1, 1, T), lambda i: (i, 0, 0)),
            pl.BlockSpec((1, 1), lambda i: (0, 0)),
            pl.BlockSpec((128, 64), lambda i: (0, 0)),
        ],
        out_shape=[
            jax.ShapeDtypeStruct((1, 1), jnp.float32),
            jax.ShapeDtypeStruct((NB, 1, T), jnp.int32),
            jax.ShapeDtypeStruct((1, 1), jnp.float32),
            jax.ShapeDtypeStruct((128, 64), jnp.float32),
        ],
    )(zbf, emb_w + emb_w, zsum, esum, lc)


_sc_mesh = plsc.VectorSubcoreMesh(core_axis_name="c", subcore_axis_name="s")


@functools.partial(
    pl.kernel, mesh=_sc_mesh,
    out_type=jax.ShapeDtypeStruct((NTS, 128), jnp.float32),
    scratch_types=[
        pltpu.VMEM((CH,), jnp.int32),
        pltpu.VMEM((CCH, 128), jnp.float32),
        pltpu.VMEM((CCH, 128), jnp.float32),
        pltpu.SemaphoreType.DMA,
        pltpu.SemaphoreType.DMA,
        pltpu.SemaphoreType.DMA,
        pltpu.SemaphoreType.DMA,
    ],
)
def _sc_gather_st(idx_hbm, emb_hbm, out_hbm,
                  idx_v, rows0, rows1, g0, g1, o0, o1):
    wid = lax.axis_index("s") * 2 + lax.axis_index("c")
    base = wid * CH
    pltpu.sync_copy(idx_hbm.at[pl.ds(base, CH)], idx_v)

    def gather(ci, rows, sem):
        return pltpu.async_copy(
            emb_hbm.at[idx_v.at[pl.ds(ci * CCH, CCH)]], rows, sem)

    def flush(ci, rows, sem):
        return pltpu.async_copy(
            rows, out_hbm.at[pl.ds(base + ci * CCH, CCH)], sem)

    gather(0, rows0, g0)

    def body(cj, _):
        e = 2 * cj
        cp_g1 = gather(e + 1, rows1, g1)
        pltpu.make_async_copy(emb_hbm.at[idx_v.at[pl.ds(e * CCH, CCH)]],
                              rows0, g0).wait()
        cp_o0 = flush(e, rows0, o0)
        cp_o0.wait()

        @pl.when(cj < NCH2 - 1)
        def _next():
            gather(e + 2, rows0, g0)

        cp_g1.wait()
        cp_o1 = flush(e + 1, rows1, o1)
        cp_o1.wait()
        return 0

    lax.fori_loop(0, NCH2, body, 0)


@functools.partial(jax.jit, static_argnames=())
def kernel(z, emb_w):
    zf = z.reshape(N_TOK, E_DIM)
    zbf = zf.astype(jnp.bfloat16)
    zsum = jnp.sum(z * z, axis=-1).reshape(SPLIT, NB, 1, T)
    esum = jnp.sum(emb_w * emb_w, axis=1).reshape(1, N_E)
    emb_pad = jnp.pad(emb_w, ((0, 0), (0, 128 - E_DIM)))

    lc = jnp.zeros((128, 65), jnp.float32)
    idx_stages, gath_stages = [], []
    loss = perp = None
    for s in range(SPLIT):
        loss, idx, perp, cnt = _tc_stage(
            zbf[s * NTS:(s + 1) * NTS], emb_w, zsum[s], esum, lc,
            last=(s == SPLIT - 1))
        lc = jnp.concatenate([jnp.broadcast_to(loss, (128, 1)), cnt], axis=1)
        idx_flat = idx.reshape(NTS)
        idx_stages.append(idx_flat)
        gath_stages.append(_sc_gather_st(idx_flat, emb_pad))

    gath = jnp.concatenate(gath_stages, axis=0)
    idx_all = jnp.concatenate(idx_stages, axis=0)
    return (loss[0, 0], gath[:, :E_DIM].reshape(z.shape), idx_all,
            perp[0, 0])


# speedup vs baseline: 1.1613x; 1.1613x over previous
import functools

import jax
import jax.numpy as jnp
from jax import lax
from jax.experimental import pallas as pl
from jax.experimental.pallas import tpu as pltpu
from jax.experimental.pallas import tpu_sc as plsc

N_E = 8192
HALF = N_E // 2
E_DIM = 32
BETA = 0.25
N_TOK = 16384
SPLIT = 1
NTS = N_TOK // SPLIT
T = 1024
NB = NTS // T
NW = 32
CH = NTS // NW
CCH = 8
NCH = CH // CCH
NCH2 = NCH // 2


def _vq_body(zbf_ref, emb_ref, zsum_ref, esum_ref, lc_ref,
             loss_ref, idx_ref, perp_ref, cnt_ref, *, last):
    pid = pl.program_id(0)

    zbf = zbf_ref[...]
    emb = emb_ref[...]

    mm = jax.lax.dot_general(
        zbf, emb, (((1,), (1,)), ((), ())),
        preferred_element_type=jnp.float32)

    zs = zsum_ref[0, 0, :][:, None]
    es = esum_ref[...]
    d = (zs + es) - 2.0 * mm

    col = jax.lax.broadcasted_iota(jnp.int32, (T, HALF), 1)
    d0 = d[:, :HALF]
    d1 = d[:, HALF:]
    m0 = jnp.min(d0, axis=1, keepdims=True)
    m1 = jnp.min(d1, axis=1, keepdims=True)
    m0b = m0.astype(jnp.bfloat16).astype(jnp.float32)
    take1 = m1 < m0b
    dsel = jnp.where(take1, d1, d0)
    msel = jnp.where(take1, m1, m0)
    isel = jnp.min(jnp.where(dsel == msel, col, 2 ** 30), axis=1)
    idx = isel + jnp.where(take1[:, 0], HALF, 0)
    idx_ref[0, 0, :] = idx
    dpick = msel[:, 0]

    hi = idx >> 6
    lo = idx & 63
    ohh = (jax.lax.broadcasted_iota(jnp.int32, (T, 128), 1)
           == hi[:, None]).astype(jnp.float32)
    ohl = (jax.lax.broadcasted_iota(jnp.int32, (T, 64), 1)
           == lo[:, None]).astype(jnp.float32)
    cnt_blk = jax.lax.dot_general(
        ohh, ohl, (((0,), (0,)), ((), ())),
        preferred_element_type=jnp.float32)

    @pl.when(pid == 0)
    def _init():
        loss_ref[...] = lc_ref[0:1, 0:1]
        cnt_ref[...] = lc_ref[:, 1:65]

    loss_ref[...] += jnp.sum(dpick)[None, None]
    cnt_ref[...] += cnt_blk

    if last:
        @pl.when(pid == NB - 1)
        def _fin():
            loss_ref[...] = loss_ref[...] * (BETA / (N_TOK * E_DIM))
            e_mean = cnt_ref[...] * (1.0 / N_TOK)
            ent = jnp.sum(e_mean * jnp.log(e_mean + 1e-10))
            perp_ref[...] = jnp.exp(-ent)[None, None]


def _tc_stage(zbf, emb_w, zsum, esum, lc, *, last):
    return pl.pallas_call(
        functools.partial(_vq_body, last=last),
        grid=(NB,),
        in_specs=[
            pl.BlockSpec((T, E_DIM), lambda i: (i, 0)),
            pl.BlockSpec((N_E, E_DIM), lambda i: (0, 0)),
            pl.BlockSpec((1, 1, T), lambda i: (i, 0, 0)),
            pl.BlockSpec((1, N_E), lambda i: (0, 0)),
            pl.BlockSpec((128, 65), lambda i: (0, 0)),
        ],
        out_specs=[
            pl.BlockSpec((1, 1), lambda i: (0, 0)),
            pl.BlockSpec((1, 1, T), lambda i: (i, 0, 0)),
            pl.BlockSpec((1, 1), lambda i: (0, 0)),
            pl.BlockSpec((128, 64), lambda i: (0, 0)),
        ],
        out_shape=[
            jax.ShapeDtypeStruct((1, 1), jnp.float32),
            jax.ShapeDtypeStruct((NB, 1, T), jnp.int32),
            jax.ShapeDtypeStruct((1, 1), jnp.float32),
            jax.ShapeDtypeStruct((128, 64), jnp.float32),
        ],
    )(zbf, emb_w, zsum, esum, lc)


_sc_mesh = plsc.VectorSubcoreMesh(core_axis_name="c", subcore_axis_name="s")


@functools.partial(
    pl.kernel, mesh=_sc_mesh,
    out_type=jax.ShapeDtypeStruct((NTS, 128), jnp.float32),
    scratch_types=[
        pltpu.VMEM((CH,), jnp.int32),
        pltpu.VMEM((CCH, 128), jnp.float32),
        pltpu.VMEM((CCH, 128), jnp.float32),
        pltpu.SemaphoreType.DMA,
        pltpu.SemaphoreType.DMA,
        pltpu.SemaphoreType.DMA,
        pltpu.SemaphoreType.DMA,
    ],
)
def _sc_gather_st(idx_hbm, emb_hbm, out_hbm,
                  idx_v, rows0, rows1, g0, g1, o0, o1):
    wid = lax.axis_index("s") * 2 + lax.axis_index("c")
    base = wid * CH
    pltpu.sync_copy(idx_hbm.at[pl.ds(base, CH)], idx_v)

    def gather(ci, rows, sem):
        return pltpu.async_copy(
            emb_hbm.at[idx_v.at[pl.ds(ci * CCH, CCH)]], rows, sem)

    def flush(ci, rows, sem):
        return pltpu.async_copy(
            rows, out_hbm.at[pl.ds(base + ci * CCH, CCH)], sem)

    gather(0, rows0, g0)

    def body(cj, _):
        e = 2 * cj
        cp_g1 = gather(e + 1, rows1, g1)
        pltpu.make_async_copy(emb_hbm.at[idx_v.at[pl.ds(e * CCH, CCH)]],
                              rows0, g0).wait()
        cp_o0 = flush(e, rows0, o0)
        cp_o0.wait()

        @pl.when(cj < NCH2 - 1)
        def _next():
            gather(e + 2, rows0, g0)

        cp_g1.wait()
        cp_o1 = flush(e + 1, rows1, o1)
        cp_o1.wait()
        return 0

    lax.fori_loop(0, NCH2, body, 0)


@functools.partial(jax.jit, static_argnames=())
def kernel(z, emb_w):
    zf = z.reshape(N_TOK, E_DIM)
    zbf = zf.astype(jnp.bfloat16)
    zsum = jnp.sum(z * z, axis=-1).reshape(SPLIT, NB, 1, T)
    esum = jnp.sum(emb_w * emb_w, axis=1).reshape(1, N_E)
    emb_pad = jnp.pad(emb_w, ((0, 0), (0, 128 - E_DIM)))

    lc = jnp.zeros((128, 65), jnp.float32)
    idx_stages, gath_stages = [], []
    loss = perp = None
    for s in range(SPLIT):
        loss, idx, perp, cnt = _tc_stage(
            zbf[s * NTS:(s + 1) * NTS], emb_w, zsum[s], esum, lc,
            last=(s == SPLIT - 1))
        lc = jnp.concatenate([jnp.broadcast_to(loss, (128, 1)), cnt], axis=1)
        idx_flat = idx.reshape(NTS)
        idx_stages.append(idx_flat)
        gath_stages.append(_sc_gather_st(idx_flat, emb_pad))

    gath = jnp.concatenate(gath_stages, axis=0)
    idx_all = jnp.concatenate(idx_stages, axis=0)
    return (loss[0, 0], gath[:, :E_DIM].reshape(z.shape), idx_all,
            perp[0, 0])
